# baseline (device time: 81405 ns/iter reference)
import jax
import jax.numpy as jnp
from jax import lax
from jax.experimental import pallas as pl
from jax.experimental.pallas import tpu as pltpu

N_DEV = 4
S = 4


def kernel(partial, gamma):
    _, M, D = partial.shape
    x = partial.reshape(M, D)
    g = gamma.reshape(1, D)
    m_per = M // N_DEV
    m_half = m_per // 2
    m_sub = m_half // S

    def body(
        x_ref, g_ref, out_ref,
        comm_r, comm_l, send_r, recv_r, send_l, recv_l,
    ):
        my = lax.axis_index("i")
        left = (my - 1) % N_DEV
        right = (my + 1) % N_DEV

        barrier_sem = pltpu.get_barrier_semaphore()
        for nbr in (left, right):
            pl.semaphore_signal(
                barrier_sem, inc=1,
                device_id=(nbr,), device_id_type=pl.DeviceIdType.MESH,
            )
        pl.semaphore_wait(barrier_sem, 2)

        def mk(comm, sends, recvs, h, k, tgt, src=None):
            rows = pl.ds(k * m_sub, m_sub)
            return pltpu.make_async_remote_copy(
                src_ref=comm.at[h, rows] if src is None else src,
                dst_ref=comm.at[h + 1, rows],
                send_sem=sends.at[h, k],
                recv_sem=recvs.at[h, k],
                device_id=(tgt,),
                device_id_type=pl.DeviceIdType.MESH,
            )


        cr = (my - 1) % N_DEV
        cl = (my + 1) % N_DEV
        descs = {}
        for k in range(S):
            src = x_ref.at[pl.ds(cr * m_per + k * m_sub, m_sub), :]
            d = mk(comm_r, send_r, recv_r, 0, k, right, src=src)
            d.start()
            descs[("r", 0, k)] = d
            src = x_ref.at[pl.ds(cl * m_per + m_half + k * m_sub, m_sub), :]
            d = mk(comm_l, send_l, recv_l, 0, k, left, src=src)
            d.start()
            descs[("l", 0, k)] = d

        for h in range(N_DEV - 1):
            c_r = (my - 2 - h) % N_DEV
            c_l = (my + 2 + h) % N_DEV
            last = h == N_DEV - 2
            for k in range(S):
                sub = slice(k * m_sub, (k + 1) * m_sub)
                loc_r = x_ref[pl.ds(c_r * m_per + k * m_sub, m_sub), :]
                loc_l = x_ref[pl.ds(c_l * m_per + m_half + k * m_sub, m_sub), :]
                if not last:
                    descs[("r", h, k)].wait_recv()
                    comm_r[h + 1, sub, :] = comm_r[h + 1, sub, :] + loc_r
                    d = mk(comm_r, send_r, recv_r, h + 1, k, right)
                    d.start()
                    descs[("r", h + 1, k)] = d

                    descs[("l", h, k)].wait_recv()
                    comm_l[h + 1, sub, :] = comm_l[h + 1, sub, :] + loc_l
                    d = mk(comm_l, send_l, recv_l, h + 1, k, left)
                    d.start()
                    descs[("l", h + 1, k)] = d
                else:
                    descs[("r", h, k)].wait_recv()
                    y = comm_r[h + 1, sub, :] + loc_r
                    inv = lax.rsqrt(
                        jnp.mean(y * y, axis=-1, keepdims=True) + 1e-6
                    )
                    out_ref[sub, :] = y * inv * g_ref[:, :]

                    descs[("l", h, k)].wait_recv()
                    y = comm_l[h + 1, sub, :] + loc_l
                    inv = lax.rsqrt(
                        jnp.mean(y * y, axis=-1, keepdims=True) + 1e-6
                    )
                    out_ref[m_half + k * m_sub : m_half + (k + 1) * m_sub, :] = (
                        y * inv * g_ref[:, :]
                    )

        for d in descs.values():
            d.wait_send()

    return pl.pallas_call(
        body,
        out_shape=jax.ShapeDtypeStruct((m_per, D), jnp.float32),
        in_specs=[
            pl.BlockSpec(memory_space=pltpu.VMEM),
            pl.BlockSpec(memory_space=pltpu.VMEM),
        ],
        out_specs=pl.BlockSpec(memory_space=pltpu.VMEM),
        scratch_shapes=[
            pltpu.VMEM((N_DEV, m_half, D), jnp.float32),
            pltpu.VMEM((N_DEV, m_half, D), jnp.float32),
            pltpu.SemaphoreType.DMA((N_DEV - 1, S)),
            pltpu.SemaphoreType.DMA((N_DEV - 1, S)),
            pltpu.SemaphoreType.DMA((N_DEV - 1, S)),
            pltpu.SemaphoreType.DMA((N_DEV - 1, S)),
        ],
        compiler_params=pltpu.CompilerParams(collective_id=0),
    )(x, g)
